# Initial kernel scaffold; baseline (speedup 1.0000x reference)
import jax
import jax.numpy as jnp
from jax.experimental import pallas as pl


def kernel(user, item, r, c, Wu, bu, Wv, bv, Wl, bl):
    raise NotImplementedError("write your pallas kernel here")


# speedup vs baseline: 1.7783x; 1.7783x over previous
import functools

import jax
import jax.numpy as jnp
from jax.experimental import pallas as pl
from jax.experimental.pallas import tpu as pltpu

_BLOCK = 1000


def _gc_block_kernel(item_ref, user_ref, ohc_ref, Wu_ref, bu_ref, Wv_ref,
                     bv_ref, Wl_ref, bl_ref, u_out_ref, v_out_ref, *,
                     num_classes, hidden):
    x_item = item_ref[...]
    x_user = user_ref[...]
    m = ohc_ref[...]
    zu = jnp.dot(x_item, Wu_ref[...], preferred_element_type=jnp.float32)
    zv = jnp.dot(x_user, Wv_ref[...], preferred_element_type=jnp.float32)
    H = hidden
    un = m[:, 0:1] * (zu[:, 0:H] + bu_ref[0:1, :])
    vn = m[:, 0:1] * (zv[:, 0:H] + bv_ref[0:1, :])
    for cc in range(1, num_classes):
        un += m[:, cc:cc + 1] * (zu[:, cc * H:(cc + 1) * H] + bu_ref[cc:cc + 1, :])
        vn += m[:, cc:cc + 1] * (zv[:, cc * H:(cc + 1) * H] + bv_ref[cc:cc + 1, :])
    hu = jnp.maximum(un, 0.0)
    hv = jnp.maximum(vn, 0.0)
    ou = jnp.dot(hu, Wl_ref[...], preferred_element_type=jnp.float32) + bl_ref[...]
    ov = jnp.dot(hv, Wl_ref[...], preferred_element_type=jnp.float32) + bl_ref[...]
    u_out_ref[...] = jnp.maximum(ou, 0.0)
    v_out_ref[...] = jnp.maximum(ov, 0.0)


def kernel(user, item, r, c, Wu, bu, Wv, bv, Wl, bl):
    N, I = user.shape
    C, H, _ = Wu.shape
    O = Wl.shape[0]
    Wu_all = jnp.transpose(Wu, (2, 0, 1)).reshape(I, C * H)
    Wv_all = jnp.transpose(Wv, (2, 0, 1)).reshape(I, C * H)
    WlT = jnp.transpose(Wl)
    ohc = (r[:, None] == jnp.arange(C, dtype=r.dtype)[None, :]).astype(
        jnp.float32) * c[:, None]
    nb = N // _BLOCK
    bs_x = pl.BlockSpec((_BLOCK, I), lambda i: (i, 0))
    bs_m = pl.BlockSpec((_BLOCK, C), lambda i: (i, 0))
    bs_W = pl.BlockSpec((I, C * H), lambda i: (0, 0))
    bs_b = pl.BlockSpec((C, H), lambda i: (0, 0))
    bs_Wl = pl.BlockSpec((H, O), lambda i: (0, 0))
    bs_bl = pl.BlockSpec((1, O), lambda i: (0, 0))
    bs_out = pl.BlockSpec((_BLOCK, O), lambda i: (i, 0))
    u_out, v_out = pl.pallas_call(
        functools.partial(_gc_block_kernel, num_classes=C, hidden=H),
        grid=(nb,),
        in_specs=[bs_x, bs_x, bs_m, bs_W, bs_b, bs_W, bs_b, bs_Wl, bs_bl],
        out_specs=[bs_out, bs_out],
        out_shape=[jax.ShapeDtypeStruct((N, O), jnp.float32)] * 2,
        compiler_params=pltpu.CompilerParams(
            dimension_semantics=("parallel",)),
    )(item, user, ohc, Wu_all, bu, Wv_all, bv, WlT, bl.reshape(1, O))
    return (u_out, v_out)
